# baseline (speedup 1.0000x reference)
import jax
import jax.numpy as jnp
from jax import lax
from jax.experimental import pallas as pl
from jax.experimental.pallas import tpu as pltpu
from jax.experimental.pallas import tpu_sc as plsc

BATCH = 16384
FIELDS = 100
EMBED_DIM = 128
VOCAB = 100000

NC = 2
NS = 16
NW = NC * NS
BPW = BATCH // NW
CH = 128
NCHUNK = BPW // CH
NV = EMBED_DIM // 16
HALF = FIELDS // 2
F16_SCALE = float(2.0 ** 112)


def _fm_body(idx_hbm, emb_hbm, lpk_hbm, bias_hbm, out_hbm,
             idx_v, idx2_v, buf0, buf1, buf2, buf3, lpk_v, bias_v, out_v,
             sem0, sem1, sem2, sem3):
    wid = lax.axis_index("s") * NC + lax.axis_index("c")
    base = wid * BPW

    pltpu.sync_copy(lpk_hbm, lpk_v)
    pltpu.sync_copy(bias_hbm, bias_v)
    bvec = bias_v[pl.ds(0, 16)]
    lanes = lax.iota(jnp.int32, 16)
    lane0 = lanes == 0
    zeros = jnp.zeros((16,), jnp.float32)
    sems = (sem0, sem1, sem2, sem3)
    bufs = (buf0, buf1, buf2, buf3)

    def fire(j, b):
        pltpu.async_copy(emb_hbm.at[idx_v.at[j]], bufs[b], sems[b])

    def wait(b):
        pltpu.make_async_copy(emb_hbm.at[idx_v.at[0]], bufs[b],
                              sems[b]).wait()

    def lin_lookup(ix):
        w = plsc.load_gather(lpk_v, [ix >> 1])
        sh = w >> ((ix & 1) << 4)
        m = sh & 0x7FFF
        sign = sh & 0x8000
        return plsc.bitcast((m << 13) | (sign << 16), jnp.float32) * F16_SCALE

    def compute(j, b, ci):
        def row_acc(r, carry):
            new = list(carry)
            for v in range(NV):
                x = bufs[b][r, pl.ds(v * 16, 16)]
                new[v] = new[v] + x
                new[NV + v] = new[NV + v] + x * x
            return tuple(new)

        accs = lax.fori_loop(0, FIELDS, row_acc, (zeros,) * (2 * NV))

        lsum = zeros
        for v in range(FIELDS // 16):
            lsum = lsum + lin_lookup(idx_v[j, pl.ds(v * 16, 16)])
        g = lin_lookup(idx_v[j, pl.ds(FIELDS - 16, 16)])
        lsum = lsum + jnp.where(lanes >= 12, g, zeros)

        t = zeros
        for v in range(NV):
            s = accs[v]
            t = t + (s * s - accs[NV + v])
        rvec = 0.5 * t + lsum + bvec
        res = jnp.full((16,), jnp.sum(rvec), jnp.float32)
        posv = jnp.full((16,), ci * CH, jnp.int32) + j
        plsc.store_scatter(out_v, [posv], res, mask=lane0)

    NB = 4
    for ci in range(NCHUNK):
        pltpu.sync_copy(idx_hbm.at[pl.ds(base + ci * CH, CH)], idx_v)
        if ci == 0:
            for b in range(NB):
                fire(b, b)

        def quad_body(p, _):
            for q in range(NB):
                j = NB * p + q
                wait(q)
                compute(j, q, ci)
                fire(j + NB, q)
            return 0

        lax.fori_loop(0, CH // NB - 1, quad_body, 0)

        if ci + 1 < NCHUNK:
            pltpu.sync_copy(
                idx_hbm.at[pl.ds(base + (ci + 1) * CH, NB)], idx2_v)
        for q in range(NB):
            j = CH - NB + q
            wait(q)
            compute(j, q, ci)
            if ci + 1 < NCHUNK:
                pltpu.async_copy(emb_hbm.at[idx2_v.at[q]], bufs[q], sems[q])

    pltpu.sync_copy(out_v, out_hbm.at[pl.ds(base, BPW)])


def kernel(interaction_pairs, emb_table, lin_table, bias):
    lin_pk = lax.bitcast_convert_type(
        lin_table.astype(jnp.float16).reshape((VOCAB // 2, 2)), jnp.int32)
    bias16 = jnp.pad(bias, (0, 15))
    mesh = plsc.VectorSubcoreMesh(core_axis_name="c", subcore_axis_name="s")
    fm = pl.kernel(
        _fm_body,
        out_type=jax.ShapeDtypeStruct((BATCH,), jnp.float32),
        mesh=mesh,
        scratch_types=[
            pltpu.VMEM((CH, FIELDS), jnp.int32),
            pltpu.VMEM((4, FIELDS), jnp.int32),
            pltpu.VMEM((FIELDS, EMBED_DIM), jnp.float32),
            pltpu.VMEM((FIELDS, EMBED_DIM), jnp.float32),
            pltpu.VMEM((FIELDS, EMBED_DIM), jnp.float32),
            pltpu.VMEM((FIELDS, EMBED_DIM), jnp.float32),
            pltpu.VMEM((VOCAB // 2,), jnp.int32),
            pltpu.VMEM((16,), jnp.float32),
            pltpu.VMEM((BPW,), jnp.float32),
            pltpu.SemaphoreType.DMA,
            pltpu.SemaphoreType.DMA,
            pltpu.SemaphoreType.DMA,
            pltpu.SemaphoreType.DMA,
        ],
        compiler_params=pltpu.CompilerParams(needs_layout_passes=False),
    )
    return fm(interaction_pairs, emb_table, lin_pk, bias16)


# speedup vs baseline: 1.0023x; 1.0023x over previous
import jax
import jax.numpy as jnp
from jax import lax
from jax.experimental import pallas as pl
from jax.experimental.pallas import tpu as pltpu
from jax.experimental.pallas import tpu_sc as plsc

BATCH = 16384
FIELDS = 100
EMBED_DIM = 128
VOCAB = 100000

NC = 2
NS = 16
NW = NC * NS
BPW = BATCH // NW
CH = 128
NCHUNK = BPW // CH
NV = EMBED_DIM // 16
F16_SCALE = float(2.0 ** 112)


def _fm_body(idx_hbm, emb_hbm, lpk_hbm, bias_hbm, out_hbm,
             idx_v, idx2_v, buf0, buf1, buf2, buf3, lpk_v, bias_v, out_v,
             sem0, sem1, sem2, sem3):
    wid = lax.axis_index("s") * NC + lax.axis_index("c")
    base = wid * BPW

    pltpu.sync_copy(lpk_hbm, lpk_v)
    pltpu.sync_copy(bias_hbm, bias_v)
    bvec = bias_v[pl.ds(0, 16)]
    lanes = lax.iota(jnp.int32, 16)
    lane0 = lanes == 0
    zeros = jnp.zeros((16,), jnp.float32)
    sems = (sem0, sem1, sem2, sem3)
    bufs = (buf0, buf1, buf2, buf3)

    def fire(j, b):
        pltpu.async_copy(emb_hbm.at[idx_v.at[j]], bufs[b], sems[b])

    def wait(b):
        pltpu.make_async_copy(emb_hbm.at[idx_v.at[0]], bufs[b],
                              sems[b]).wait()

    def lin_lookup(ix):
        w = plsc.load_gather(lpk_v, [ix >> 1])
        sh = w >> ((ix & 1) << 4)
        m = sh & 0x7FFF
        sign = sh & 0x8000
        return plsc.bitcast((m << 13) | (sign << 16), jnp.float32) * F16_SCALE

    def compute(j, b, ci):
        def row_acc(r, carry):
            new = list(carry)
            for v in range(NV):
                x = bufs[b][r, pl.ds(v * 16, 16)]
                new[v] = new[v] + x
                new[NV + v] = new[NV + v] + x * x
            return tuple(new)

        accs = lax.fori_loop(0, FIELDS, row_acc, (zeros,) * (2 * NV))

        lsum = zeros
        for v in range(FIELDS // 16):
            lsum = lsum + lin_lookup(idx_v[j, pl.ds(v * 16, 16)])
        g = lin_lookup(idx_v[j, pl.ds(FIELDS - 16, 16)])
        lsum = lsum + jnp.where(lanes >= 12, g, zeros)

        t = zeros
        for v in range(NV):
            s = accs[v]
            t = t + (s * s - accs[NV + v])
        rvec = 0.5 * t + lsum + bvec
        res = jnp.full((16,), jnp.sum(rvec), jnp.float32)
        posv = jnp.full((16,), ci * CH, jnp.int32) + j
        plsc.store_scatter(out_v, [posv], res, mask=lane0)

    NB = 4
    for ci in range(NCHUNK):
        pltpu.sync_copy(idx_hbm.at[pl.ds(base + ci * CH, CH)], idx_v)
        if ci == 0:
            for b in range(NB):
                fire(b, b)

        def quad_body(p, _):
            for q in range(NB):
                j = NB * p + q
                wait(q)
                compute(j, q, ci)
                fire(j + NB, q)
            return 0

        lax.fori_loop(0, CH // NB - 1, quad_body, 0)

        if ci + 1 < NCHUNK:
            pltpu.sync_copy(
                idx_hbm.at[pl.ds(base + (ci + 1) * CH, NB)], idx2_v)
        for q in range(NB):
            j = CH - NB + q
            wait(q)
            compute(j, q, ci)
            if ci + 1 < NCHUNK:
                pltpu.async_copy(emb_hbm.at[idx2_v.at[q]], bufs[q], sems[q])

    pltpu.sync_copy(out_v, out_hbm.at[pl.ds(base, BPW)])


def kernel(interaction_pairs, emb_table, lin_table, bias):
    lin_pk = lax.bitcast_convert_type(
        lin_table.astype(jnp.float16).reshape((VOCAB // 2, 2)), jnp.int32)
    bias16 = jnp.pad(bias, (0, 15))
    mesh = plsc.VectorSubcoreMesh(core_axis_name="c", subcore_axis_name="s")
    fm = pl.kernel(
        _fm_body,
        out_type=jax.ShapeDtypeStruct((BATCH,), jnp.float32),
        mesh=mesh,
        scratch_types=[
            pltpu.VMEM((CH, FIELDS), jnp.int32),
            pltpu.VMEM((4, FIELDS), jnp.int32),
            pltpu.VMEM((FIELDS, EMBED_DIM), jnp.float32),
            pltpu.VMEM((FIELDS, EMBED_DIM), jnp.float32),
            pltpu.VMEM((FIELDS, EMBED_DIM), jnp.float32),
            pltpu.VMEM((FIELDS, EMBED_DIM), jnp.float32),
            pltpu.VMEM((VOCAB // 2,), jnp.int32),
            pltpu.VMEM((16,), jnp.float32),
            pltpu.VMEM((BPW,), jnp.float32),
            pltpu.SemaphoreType.DMA,
            pltpu.SemaphoreType.DMA,
            pltpu.SemaphoreType.DMA,
            pltpu.SemaphoreType.DMA,
        ],
        compiler_params=pltpu.CompilerParams(needs_layout_passes=False),
    )
    return fm(interaction_pairs, emb_table, lin_pk, bias16)
